# baseline (speedup 1.0000x reference)
import functools

import jax
import jax.numpy as jnp
from jax import lax
from jax.experimental import pallas as pl
from jax.experimental.pallas import tpu as pltpu
from jax.experimental.pallas import tpu_sc as plsc

B = 1024
S = 96
P = 24
DIN = 64
H = 256
E = 8
K = 4
L = 4
NH = 4
DH = H // NH
DFF = 4 * H


_BTA = 16
_RA = _BTA * S


def _encoder_body(x_ref, pos_ref, wp_ref, bp_ref, wc1_ref, bc1_ref,
                  wc2_ref, bc2_ref, wq_ref, bq_ref, wk_ref, bk_ref,
                  wv_ref, bv_ref, wo_ref, bo_ref, g_ref, b_ref, out_ref):
    f32 = jnp.float32
    t_iota = lax.broadcasted_iota(jnp.int32, (_RA, 1), 0) % S
    h = x_ref[...] @ wp_ref[...] + bp_ref[...] + pos_ref[...]

    def conv(u, wr, br):
        z0 = u @ wr[0]
        z1 = u @ wr[1]
        z2 = u @ wr[2]
        c = z1 + br[...]
        down = jnp.concatenate([jnp.zeros((1, H), f32), z0[:-1, :]], axis=0)
        c = c + jnp.where(t_iota > 0, down, 0.0)
        up = jnp.concatenate([z2[1:, :], jnp.zeros((1, H), f32)], axis=0)
        c = c + jnp.where(t_iota < S - 1, up, 0.0)
        return c

    c = jax.nn.gelu(conv(h, wc1_ref, bc1_ref))
    c = conv(c, wc2_ref, bc2_ref)
    h = h + c

    q = h @ wq_ref[...] + bq_ref[...]
    k = h @ wk_ref[...] + bk_ref[...]
    v = h @ wv_ref[...] + bv_ref[...]
    att = bo_ref[...]
    for n in range(NH):
        sl = slice(n * DH, (n + 1) * DH)
        qn = q[:, sl].reshape(_BTA, S, DH)
        kn = k[:, sl].reshape(_BTA, S, DH)
        vn = v[:, sl].reshape(_BTA, S, DH)
        s = lax.dot_general(qn, kn, (((2,), (2,)), ((0,), (0,))))
        a = jax.nn.softmax(s * (1.0 / 8.0), axis=-1)
        on = lax.dot_general(a, vn, (((2,), (1,)), ((0,), (0,))))
        att = att + on.reshape(_RA, DH) @ wo_ref[sl, :]
    z = h + att
    mu = jnp.mean(z, axis=-1, keepdims=True)
    var = jnp.mean((z - mu) ** 2, axis=-1, keepdims=True)
    out_ref[...] = g_ref[...] * (z - mu) / jnp.sqrt(var + 1e-5) + b_ref[...]


def _encoder(x2, pos_t, wp, bp, wc1t, bc1, wc2t, bc2, wq, bq, wk, bk,
             wv, bv, wo, bo, g, b):
    n = B // _BTA
    row = lambda i: (i, 0)
    const = lambda i: (0, 0)
    const3 = lambda i: (0, 0, 0)
    return pl.pallas_call(
        _encoder_body,
        grid=(n,),
        in_specs=[
            pl.BlockSpec((_RA, DIN), row),
            pl.BlockSpec((_RA, H), const),
            pl.BlockSpec((DIN, H), const),
            pl.BlockSpec((1, H), const),
            pl.BlockSpec((3, H, H), const3),
            pl.BlockSpec((1, H), const),
            pl.BlockSpec((3, H, H), const3),
            pl.BlockSpec((1, H), const),
            pl.BlockSpec((H, H), const),
            pl.BlockSpec((1, H), const),
            pl.BlockSpec((H, H), const),
            pl.BlockSpec((1, H), const),
            pl.BlockSpec((H, H), const),
            pl.BlockSpec((1, H), const),
            pl.BlockSpec((H, H), const),
            pl.BlockSpec((1, H), const),
            pl.BlockSpec((1, H), const),
            pl.BlockSpec((1, H), const),
        ],
        out_specs=pl.BlockSpec((_RA, H), row),
        out_shape=jax.ShapeDtypeStruct((B * S, H), jnp.float32),
    )(x2, pos_t, wp, bp, wc1t, bc1, wc2t, bc2, wq, bq, wk, bk, wv, bv,
      wo, bo, g, b)


_KC = 2048
_NK = (S * H) // _KC


def _pool_body(hs_ref, wa1_ref, ba1_ref, wa2_ref, ba2_ref, wr_ref, br_ref,
               out_ref, lg_ref, acc):
    k = pl.program_id(0)

    @pl.when(k == 0)
    def _():
        acc[...] = jnp.zeros_like(acc)

    acc[...] += hs_ref[...] @ wa1_ref[...]

    @pl.when(k == _NK - 1)
    def _():
        u = jax.nn.gelu(acc[...] + ba1_ref[...])
        h0 = u @ wa2_ref[...] + ba2_ref[...]
        out_ref[...] = h0
        lg_ref[...] = h0 @ wr_ref[...] + br_ref[...]


def _pool(hs_big, wa1, ba1, wa2, ba2, wr0, br0):
    c2 = lambda k: (0, 0)
    return pl.pallas_call(
        _pool_body,
        grid=(_NK,),
        in_specs=[
            pl.BlockSpec((B, _KC), lambda k: (0, k)),
            pl.BlockSpec((_KC, 2 * H), lambda k: (k, 0)),
            pl.BlockSpec((1, 2 * H), c2),
            pl.BlockSpec((2 * H, H), c2),
            pl.BlockSpec((1, H), c2),
            pl.BlockSpec((H, E), c2),
            pl.BlockSpec((1, E), c2),
        ],
        out_specs=[pl.BlockSpec((B, H), c2), pl.BlockSpec((B, E), c2)],
        out_shape=[jax.ShapeDtypeStruct((B, H), jnp.float32),
                   jax.ShapeDtypeStruct((B, E), jnp.float32)],
        scratch_shapes=[pltpu.VMEM((B, 2 * H), jnp.float32)],
    )(hs_big, wa1, ba1, wa2, ba2, wr0, br0)


_NW = 32
_TPW = B // _NW
_LN = 16


def _route_body(lt_hbm, out_hbm, lt_v, out_v):
    wid = lax.axis_index("s") * 2 + lax.axis_index("c")
    pltpu.sync_copy(lt_hbm.at[wid], lt_v)
    for ch in range(_TPW // _LN):
        sl = pl.ds(ch * _LN, _LN)
        v = [lt_v[e, sl] for e in range(E)]
        m = v[0]
        for e in range(1, E):
            m = jnp.maximum(m, v[e])
        ex = [jnp.exp(u - m) for u in v]
        tot = ex[0]
        for e in range(1, E):
            tot = tot + ex[e]
        p = [u / tot for u in ex]
        active = [jnp.full((_LN,), 1.0, jnp.float32) for _ in range(E)]
        acc = [jnp.zeros((_LN,), jnp.float32) for _ in range(E)]
        denom = jnp.zeros((_LN,), jnp.float32)
        for _ in range(K):
            cur = [p[e] * active[e] + (active[e] - 1.0) for e in range(E)]
            mv = cur[0]
            for e in range(1, E):
                mv = jnp.maximum(mv, cur[e])
            found = jnp.zeros((_LN,), jnp.float32)
            for e in range(E):
                eqf = jnp.where(cur[e] == mv, 1.0, 0.0)
                self_f = eqf * (1.0 - found)
                add = self_f * p[e]
                acc[e] = acc[e] + add
                denom = denom + add
                found = found + self_f
                active[e] = active[e] * (1.0 - self_f)
        for e in range(E):
            out_v[e, sl] = acc[e] / denom
    pltpu.sync_copy(out_v, out_hbm.at[wid])


def _route(ltw):
    mesh = plsc.VectorSubcoreMesh(core_axis_name="c", subcore_axis_name="s")
    return pl.kernel(
        _route_body,
        mesh=mesh,
        out_type=jax.ShapeDtypeStruct((_NW, E, _TPW), jnp.float32),
        scratch_types=[pltpu.VMEM((E, _TPW), jnp.float32),
                       pltpu.VMEM((E, _TPW), jnp.float32)],
    )(ltw)


def _moe_body(h_ref, comb_ref, we1_ref, be1_ref, we2_ref, be2_ref,
              g_ref, b_ref, wrn_ref, brn_ref, out_ref, lg_ref, moe):
    e = pl.program_id(0)
    lane = lax.broadcasted_iota(jnp.int32, (B, E), 1)

    @pl.when(e == 0)
    def _():
        moe[...] = jnp.zeros_like(moe)

    eh = jax.nn.gelu(h_ref[...] @ we1_ref[0] + be1_ref[0])
    eo = eh @ we2_ref[0] + be2_ref[0]
    ce = jnp.sum(jnp.where(lane == e, comb_ref[...], 0.0), axis=-1,
                 keepdims=True)
    moe[...] += ce * eo

    @pl.when(e == E - 1)
    def _():
        z = h_ref[...] + moe[...]
        mu = jnp.mean(z, axis=-1, keepdims=True)
        var = jnp.mean((z - mu) ** 2, axis=-1, keepdims=True)
        hn = g_ref[...] * (z - mu) / jnp.sqrt(var + 1e-5) + b_ref[...]
        out_ref[...] = hn
        lg_ref[...] = hn @ wrn_ref[...] + brn_ref[...]


def _moe_layer(h, comb, we1, be1, we2, be2, g, b, wr_next, br_next):
    c2 = lambda e: (0, 0)
    return pl.pallas_call(
        _moe_body,
        grid=(E,),
        in_specs=[
            pl.BlockSpec((B, H), c2),
            pl.BlockSpec((B, E), c2),
            pl.BlockSpec((1, H, DFF), lambda e: (e, 0, 0)),
            pl.BlockSpec((1, 1, DFF), lambda e: (e, 0, 0)),
            pl.BlockSpec((1, DFF, H), lambda e: (e, 0, 0)),
            pl.BlockSpec((1, 1, H), lambda e: (e, 0, 0)),
            pl.BlockSpec((1, H), c2),
            pl.BlockSpec((1, H), c2),
            pl.BlockSpec((H, E), c2),
            pl.BlockSpec((1, E), c2),
        ],
        out_specs=[pl.BlockSpec((B, H), c2), pl.BlockSpec((B, E), c2)],
        out_shape=[jax.ShapeDtypeStruct((B, H), jnp.float32),
                   jax.ShapeDtypeStruct((B, E), jnp.float32)],
        scratch_shapes=[pltpu.VMEM((B, H), jnp.float32)],
    )(h, comb, we1, be1, we2, be2, g, b, wr_next, br_next)


_BTD = 256


def _head_body(h_ref, wexp_ref, bexp_ref, wh1_ref, bh1_ref, wh2_ref,
               bh2_ref, out_ref):
    d = jax.nn.gelu(h_ref[...] @ wexp_ref[...] + bexp_ref[...])
    cols = []
    for p in range(P):
        dp = d[:, p * H:(p + 1) * H]
        t = jax.nn.gelu(dp @ wh1_ref[...] + bh1_ref[...])
        op = jnp.sum(t * wh2_ref[...], axis=-1, keepdims=True) + bh2_ref[...]
        cols.append(op)
    out_ref[...] = jnp.concatenate(cols, axis=1)


def _head(h, wexp, bexp, wh1, bh1, wh2t, bh2):
    n = B // _BTD
    c2 = lambda i: (0, 0)
    return pl.pallas_call(
        _head_body,
        grid=(n,),
        in_specs=[
            pl.BlockSpec((_BTD, H), lambda i: (i, 0)),
            pl.BlockSpec((H, P * H), c2),
            pl.BlockSpec((1, P * H), c2),
            pl.BlockSpec((H, H // 2), c2),
            pl.BlockSpec((1, H // 2), c2),
            pl.BlockSpec((1, H // 2), c2),
            pl.BlockSpec((1, 1), c2),
        ],
        out_specs=pl.BlockSpec((_BTD, P), lambda i: (i, 0)),
        out_shape=jax.ShapeDtypeStruct((B, P), jnp.float32),
    )(h, wexp, bexp, wh1, bh1, wh2t, bh2)


def kernel(x, pos_emb, Wp, bp, Wc1, bc1, Wc2, bc2, Wq, bq, Wk, bk, Wv, bv,
           Wo, bo, ln1g, ln1b, Wa1, ba1, Wa2, ba2, Wr, br, We1, be1, We2,
           be2, lng, lnb, Wexp, bexp, Wh1, bh1, Wh2, bh2):
    x2 = x.reshape(B * S, DIN)
    pos_t = jnp.tile(pos_emb[0], (_BTA, 1))
    wc1t = jnp.transpose(Wc1, (2, 1, 0))
    wc2t = jnp.transpose(Wc2, (2, 1, 0))
    r1 = lambda a: a.reshape(1, -1)

    hs = _encoder(x2, pos_t, Wp, r1(bp), wc1t, r1(bc1), wc2t, r1(bc2),
                  Wq, r1(bq), Wk, r1(bk), Wv, r1(bv), Wo, r1(bo),
                  r1(ln1g), r1(ln1b))
    h, lg = _pool(hs.reshape(B, S * H), Wa1, r1(ba1), Wa2, r1(ba2),
                  Wr[0], r1(br[0]))
    for l in range(L):
        ltw = lg.reshape(_NW, _TPW, E).transpose(0, 2, 1)
        cw = _route(ltw)
        comb = cw.transpose(0, 2, 1).reshape(B, E)
        ln = (l + 1) % L
        h, lg = _moe_layer(h, comb, We1[l], be1[l].reshape(E, 1, DFF),
                           We2[l], be2[l].reshape(E, 1, H), r1(lng[l]),
                           r1(lnb[l]), Wr[ln], r1(br[ln]))
    out = _head(h, Wexp, r1(bexp), Wh1, r1(bh1), Wh2.reshape(1, H // 2),
                bh2.reshape(1, 1))
    return out


# speedup vs baseline: 1.0007x; 1.0007x over previous
import functools

import jax
import jax.numpy as jnp
from jax import lax
from jax.experimental import pallas as pl
from jax.experimental.pallas import tpu as pltpu
from jax.experimental.pallas import tpu_sc as plsc

B = 1024
S = 96
P = 24
DIN = 64
H = 256
E = 8
K = 4
L = 4
NH = 4
DH = H // NH
DFF = 4 * H


_BTA = 16
_RA = _BTA * S


def _encoder_body(x_ref, pos_ref, wp_ref, bp_ref, wc1_ref, bc1_ref,
                  wc2_ref, bc2_ref, wq_ref, bq_ref, wk_ref, bk_ref,
                  wv_ref, bv_ref, wo_ref, bo_ref, g_ref, b_ref, out_ref):
    f32 = jnp.float32
    t_iota = lax.broadcasted_iota(jnp.int32, (_RA, 1), 0) % S
    h = x_ref[...] @ wp_ref[...] + bp_ref[...] + pos_ref[...]

    def conv(u, wr, br):
        z0 = u @ wr[0]
        z1 = u @ wr[1]
        z2 = u @ wr[2]
        c = z1 + br[...]
        down = jnp.concatenate([jnp.zeros((1, H), f32), z0[:-1, :]], axis=0)
        c = c + jnp.where(t_iota > 0, down, 0.0)
        up = jnp.concatenate([z2[1:, :], jnp.zeros((1, H), f32)], axis=0)
        c = c + jnp.where(t_iota < S - 1, up, 0.0)
        return c

    c = jax.nn.gelu(conv(h, wc1_ref, bc1_ref))
    c = conv(c, wc2_ref, bc2_ref)
    h = h + c

    q = h @ wq_ref[...] + bq_ref[...]
    k = h @ wk_ref[...] + bk_ref[...]
    v = h @ wv_ref[...] + bv_ref[...]
    att = bo_ref[...]
    for n in range(NH):
        sl = slice(n * DH, (n + 1) * DH)
        qn = q[:, sl].reshape(_BTA, S, DH)
        kn = k[:, sl].reshape(_BTA, S, DH)
        vn = v[:, sl].reshape(_BTA, S, DH)
        s = lax.dot_general(qn, kn, (((2,), (2,)), ((0,), (0,))))
        a = jax.nn.softmax(s * (1.0 / 8.0), axis=-1)
        on = lax.dot_general(a, vn, (((2,), (1,)), ((0,), (0,))))
        att = att + on.reshape(_RA, DH) @ wo_ref[sl, :]
    z = h + att
    mu = jnp.mean(z, axis=-1, keepdims=True)
    var = jnp.mean((z - mu) ** 2, axis=-1, keepdims=True)
    hs = g_ref[...] * (z - mu) / jnp.sqrt(var + 1e-5) + b_ref[...]
    out_ref[...] = hs.astype(jnp.bfloat16)


def _encoder(x2, pos_t, wp, bp, wc1t, bc1, wc2t, bc2, wq, bq, wk, bk,
             wv, bv, wo, bo, g, b):
    n = B // _BTA
    row = lambda i: (i, 0)
    const = lambda i: (0, 0)
    const3 = lambda i: (0, 0, 0)
    return pl.pallas_call(
        _encoder_body,
        grid=(n,),
        in_specs=[
            pl.BlockSpec((_RA, DIN), row),
            pl.BlockSpec((_RA, H), const),
            pl.BlockSpec((DIN, H), const),
            pl.BlockSpec((1, H), const),
            pl.BlockSpec((3, H, H), const3),
            pl.BlockSpec((1, H), const),
            pl.BlockSpec((3, H, H), const3),
            pl.BlockSpec((1, H), const),
            pl.BlockSpec((H, H), const),
            pl.BlockSpec((1, H), const),
            pl.BlockSpec((H, H), const),
            pl.BlockSpec((1, H), const),
            pl.BlockSpec((H, H), const),
            pl.BlockSpec((1, H), const),
            pl.BlockSpec((H, H), const),
            pl.BlockSpec((1, H), const),
            pl.BlockSpec((1, H), const),
            pl.BlockSpec((1, H), const),
        ],
        out_specs=pl.BlockSpec((_RA, H), row),
        out_shape=jax.ShapeDtypeStruct((B * S, H), jnp.bfloat16),
    )(x2, pos_t, wp, bp, wc1t, bc1, wc2t, bc2, wq, bq, wk, bk, wv, bv,
      wo, bo, g, b)


_KC = 2048
_NK = (S * H) // _KC


def _pool_body(hs_ref, wa1_ref, ba1_ref, wa2_ref, ba2_ref, wr_ref, br_ref,
               out_ref, lg_ref, acc):
    k = pl.program_id(0)

    @pl.when(k == 0)
    def _():
        acc[...] = jnp.zeros_like(acc)

    acc[...] += jnp.dot(hs_ref[...], wa1_ref[...],
                        preferred_element_type=jnp.float32)

    @pl.when(k == _NK - 1)
    def _():
        u = jax.nn.gelu(acc[...] + ba1_ref[...])
        h0 = u @ wa2_ref[...] + ba2_ref[...]
        out_ref[...] = h0
        lg_ref[...] = h0 @ wr_ref[...] + br_ref[...]


def _pool(hs_big, wa1, ba1, wa2, ba2, wr0, br0):
    c2 = lambda k: (0, 0)
    return pl.pallas_call(
        _pool_body,
        grid=(_NK,),
        in_specs=[
            pl.BlockSpec((B, _KC), lambda k: (0, k)),
            pl.BlockSpec((_KC, 2 * H), lambda k: (k, 0)),
            pl.BlockSpec((1, 2 * H), c2),
            pl.BlockSpec((2 * H, H), c2),
            pl.BlockSpec((1, H), c2),
            pl.BlockSpec((H, E), c2),
            pl.BlockSpec((1, E), c2),
        ],
        out_specs=[pl.BlockSpec((B, H), c2), pl.BlockSpec((B, E), c2)],
        out_shape=[jax.ShapeDtypeStruct((B, H), jnp.float32),
                   jax.ShapeDtypeStruct((B, E), jnp.float32)],
        scratch_shapes=[pltpu.VMEM((B, 2 * H), jnp.float32)],
    )(hs_big, wa1, ba1, wa2, ba2, wr0, br0)


_NW = 32
_TPW = B // _NW
_LN = 16


def _route_body(lt_hbm, out_hbm, lt_v, out_v):
    wid = lax.axis_index("s") * 2 + lax.axis_index("c")
    pltpu.sync_copy(lt_hbm.at[wid], lt_v)
    for ch in range(_TPW // _LN):
        sl = pl.ds(ch * _LN, _LN)
        v = [lt_v[e, sl] for e in range(E)]
        m = v[0]
        for e in range(1, E):
            m = jnp.maximum(m, v[e])
        ex = [jnp.exp(u - m) for u in v]
        tot = ex[0]
        for e in range(1, E):
            tot = tot + ex[e]
        p = [u / tot for u in ex]
        active = [jnp.full((_LN,), 1.0, jnp.float32) for _ in range(E)]
        acc = [jnp.zeros((_LN,), jnp.float32) for _ in range(E)]
        denom = jnp.zeros((_LN,), jnp.float32)
        for _ in range(K):
            cur = [p[e] * active[e] + (active[e] - 1.0) for e in range(E)]
            mv = cur[0]
            for e in range(1, E):
                mv = jnp.maximum(mv, cur[e])
            found = jnp.zeros((_LN,), jnp.float32)
            for e in range(E):
                eqf = jnp.where(cur[e] == mv, 1.0, 0.0)
                self_f = eqf * (1.0 - found)
                add = self_f * p[e]
                acc[e] = acc[e] + add
                denom = denom + add
                found = found + self_f
                active[e] = active[e] * (1.0 - self_f)
        for e in range(E):
            out_v[e, sl] = acc[e] / denom
    pltpu.sync_copy(out_v, out_hbm.at[wid])


def _route(ltw):
    mesh = plsc.VectorSubcoreMesh(core_axis_name="c", subcore_axis_name="s")
    return pl.kernel(
        _route_body,
        mesh=mesh,
        out_type=jax.ShapeDtypeStruct((_NW, E, _TPW), jnp.float32),
        scratch_types=[pltpu.VMEM((E, _TPW), jnp.float32),
                       pltpu.VMEM((E, _TPW), jnp.float32)],
    )(ltw)


def _moe_body(h_ref, comb_ref, we1_ref, be1_ref, we2_ref, be2_ref,
              g_ref, b_ref, wrn_ref, brn_ref, out_ref, lg_ref, moe):
    e = pl.program_id(0)
    lane = lax.broadcasted_iota(jnp.int32, (B, E), 1)

    @pl.when(e == 0)
    def _():
        moe[...] = jnp.zeros_like(moe)

    hb = h_ref[...].astype(jnp.bfloat16)
    eh = jax.nn.gelu(jnp.dot(hb, we1_ref[0],
                             preferred_element_type=jnp.float32) + be1_ref[0])
    eo = jnp.dot(eh.astype(jnp.bfloat16), we2_ref[0],
                 preferred_element_type=jnp.float32) + be2_ref[0]
    ce = jnp.sum(jnp.where(lane == e, comb_ref[...], 0.0), axis=-1,
                 keepdims=True)
    moe[...] += ce * eo

    @pl.when(e == E - 1)
    def _():
        z = h_ref[...] + moe[...]
        mu = jnp.mean(z, axis=-1, keepdims=True)
        var = jnp.mean((z - mu) ** 2, axis=-1, keepdims=True)
        hn = g_ref[...] * (z - mu) / jnp.sqrt(var + 1e-5) + b_ref[...]
        out_ref[...] = hn
        lg_ref[...] = hn @ wrn_ref[...] + brn_ref[...]


def _moe_layer(h, comb, we1, be1, we2, be2, g, b, wr_next, br_next):
    c2 = lambda e: (0, 0)
    return pl.pallas_call(
        _moe_body,
        grid=(E,),
        in_specs=[
            pl.BlockSpec((B, H), c2),
            pl.BlockSpec((B, E), c2),
            pl.BlockSpec((1, H, DFF), lambda e: (e, 0, 0)),
            pl.BlockSpec((1, 1, DFF), lambda e: (e, 0, 0)),
            pl.BlockSpec((1, DFF, H), lambda e: (e, 0, 0)),
            pl.BlockSpec((1, 1, H), lambda e: (e, 0, 0)),
            pl.BlockSpec((1, H), c2),
            pl.BlockSpec((1, H), c2),
            pl.BlockSpec((H, E), c2),
            pl.BlockSpec((1, E), c2),
        ],
        out_specs=[pl.BlockSpec((B, H), c2), pl.BlockSpec((B, E), c2)],
        out_shape=[jax.ShapeDtypeStruct((B, H), jnp.float32),
                   jax.ShapeDtypeStruct((B, E), jnp.float32)],
        scratch_shapes=[pltpu.VMEM((B, H), jnp.float32)],
    )(h, comb, we1, be1, we2, be2, g, b, wr_next, br_next)


_BTD = 256


def _head_body(h_ref, wexp_ref, bexp_ref, wh1_ref, bh1_ref, wh2_ref,
               bh2_ref, out_ref):
    hb = h_ref[...].astype(jnp.bfloat16)
    d = jax.nn.gelu(jnp.dot(hb, wexp_ref[...],
                            preferred_element_type=jnp.float32)
                    + bexp_ref[...])
    d2 = d.reshape(_BTD * P, H)
    t = jax.nn.gelu(jnp.dot(d2.astype(jnp.bfloat16), wh1_ref[...],
                            preferred_element_type=jnp.float32)
                    + bh1_ref[...])
    out_ref[...] = (jnp.sum(t * wh2_ref[...], axis=-1, keepdims=True)
                    + bh2_ref[...])


def _head(h, wexp, bexp, wh1, bh1, wh2t, bh2):
    n = B // _BTD
    c2 = lambda i: (0, 0)
    return pl.pallas_call(
        _head_body,
        grid=(n,),
        in_specs=[
            pl.BlockSpec((_BTD, H), lambda i: (i, 0)),
            pl.BlockSpec((H, P * H), c2),
            pl.BlockSpec((1, P * H), c2),
            pl.BlockSpec((H, H // 2), c2),
            pl.BlockSpec((1, H // 2), c2),
            pl.BlockSpec((1, H // 2), c2),
            pl.BlockSpec((1, 1), c2),
        ],
        out_specs=pl.BlockSpec((_BTD * P, 1), lambda i: (i, 0)),
        out_shape=jax.ShapeDtypeStruct((B * P, 1), jnp.float32),
    )(h, wexp, bexp, wh1, bh1, wh2t, bh2)


def kernel(x, pos_emb, Wp, bp, Wc1, bc1, Wc2, bc2, Wq, bq, Wk, bk, Wv, bv,
           Wo, bo, ln1g, ln1b, Wa1, ba1, Wa2, ba2, Wr, br, We1, be1, We2,
           be2, lng, lnb, Wexp, bexp, Wh1, bh1, Wh2, bh2):
    x2 = x.reshape(B * S, DIN)
    pos_t = jnp.tile(pos_emb[0], (_BTA, 1))
    wc1t = jnp.transpose(Wc1, (2, 1, 0))
    wc2t = jnp.transpose(Wc2, (2, 1, 0))
    r1 = lambda a: a.reshape(1, -1)

    hs = _encoder(x2, pos_t, Wp, r1(bp), wc1t, r1(bc1), wc2t, r1(bc2),
                  Wq, r1(bq), Wk, r1(bk), Wv, r1(bv), Wo, r1(bo),
                  r1(ln1g), r1(ln1b))
    bf = lambda a: a.astype(jnp.bfloat16)
    h, lg = _pool(hs.reshape(B, S * H), bf(Wa1), r1(ba1), Wa2, r1(ba2),
                  Wr[0], r1(br[0]))
    for l in range(L):
        ltw = lg.reshape(_NW, _TPW, E).transpose(0, 2, 1)
        cw = _route(ltw)
        comb = cw.transpose(0, 2, 1).reshape(B, E)
        ln = (l + 1) % L
        h, lg = _moe_layer(h, comb, bf(We1[l]), be1[l].reshape(E, 1, DFF),
                           bf(We2[l]), be2[l].reshape(E, 1, H), r1(lng[l]),
                           r1(lnb[l]), Wr[ln], r1(br[ln]))
    out = _head(h, bf(Wexp), r1(bexp), bf(Wh1), r1(bh1),
                Wh2.reshape(1, H // 2), bh2.reshape(1, 1))
    return out.reshape(B, P)


# speedup vs baseline: 1.0370x; 1.0362x over previous
import functools

import jax
import jax.numpy as jnp
from jax import lax
from jax.experimental import pallas as pl
from jax.experimental.pallas import tpu as pltpu
from jax.experimental.pallas import tpu_sc as plsc

B = 1024
S = 96
P = 24
DIN = 64
H = 256
E = 8
K = 4
L = 4
NH = 4
DH = H // NH
DFF = 4 * H


_BTA = 16
_RA = _BTA * S


def _bdot(a, b):
    return jnp.dot(a.astype(jnp.bfloat16), b.astype(jnp.bfloat16),
                   preferred_element_type=jnp.float32)


def _encoder_body(x_ref, pos_ref, wp_ref, bp_ref, wc1_ref, bc1_ref,
                  wc2_ref, bc2_ref, wq_ref, bq_ref, wk_ref, bk_ref,
                  wv_ref, bv_ref, wo_ref, bo_ref, g_ref, b_ref, out_ref):
    f32 = jnp.float32
    t_iota = lax.broadcasted_iota(jnp.int32, (_RA, 1), 0) % S
    h = x_ref[...] @ wp_ref[...] + bp_ref[...] + pos_ref[...]

    def conv(u, wr, br):
        z0 = u @ wr[0]
        z1 = u @ wr[1]
        z2 = u @ wr[2]
        c = z1 + br[...]
        down = jnp.concatenate([jnp.zeros((1, H), f32), z0[:-1, :]], axis=0)
        c = c + jnp.where(t_iota > 0, down, 0.0)
        up = jnp.concatenate([z2[1:, :], jnp.zeros((1, H), f32)], axis=0)
        c = c + jnp.where(t_iota < S - 1, up, 0.0)
        return c

    c = jax.nn.gelu(conv(h, wc1_ref, bc1_ref))
    c = conv(c, wc2_ref, bc2_ref)
    h = h + c

    q = h @ wq_ref[...] + bq_ref[...]
    k = h @ wk_ref[...] + bk_ref[...]
    v = h @ wv_ref[...] + bv_ref[...]
    att = bo_ref[...]
    for n in range(NH):
        sl = slice(n * DH, (n + 1) * DH)
        qn = q[:, sl].reshape(_BTA, S, DH)
        kn = k[:, sl].reshape(_BTA, S, DH)
        vn = v[:, sl].reshape(_BTA, S, DH)
        s = lax.dot_general(qn, kn, (((2,), (2,)), ((0,), (0,))))
        a = jax.nn.softmax(s * (1.0 / 8.0), axis=-1)
        on = lax.dot_general(a, vn, (((2,), (1,)), ((0,), (0,))))
        att = att + on.reshape(_RA, DH) @ wo_ref[sl, :]
    z = h + att
    mu = jnp.mean(z, axis=-1, keepdims=True)
    var = jnp.mean((z - mu) ** 2, axis=-1, keepdims=True)
    hs = g_ref[...] * (z - mu) / jnp.sqrt(var + 1e-5) + b_ref[...]
    out_ref[...] = hs.astype(jnp.bfloat16)


def _encoder(x2, pos_t, wp, bp, wc1t, bc1, wc2t, bc2, wq, bq, wk, bk,
             wv, bv, wo, bo, g, b):
    n = B // _BTA
    row = lambda i: (i, 0)
    const = lambda i: (0, 0)
    const3 = lambda i: (0, 0, 0)
    return pl.pallas_call(
        _encoder_body,
        grid=(n,),
        in_specs=[
            pl.BlockSpec((_RA, DIN), row),
            pl.BlockSpec((_RA, H), const),
            pl.BlockSpec((DIN, H), const),
            pl.BlockSpec((1, H), const),
            pl.BlockSpec((3, H, H), const3),
            pl.BlockSpec((1, H), const),
            pl.BlockSpec((3, H, H), const3),
            pl.BlockSpec((1, H), const),
            pl.BlockSpec((H, H), const),
            pl.BlockSpec((1, H), const),
            pl.BlockSpec((H, H), const),
            pl.BlockSpec((1, H), const),
            pl.BlockSpec((H, H), const),
            pl.BlockSpec((1, H), const),
            pl.BlockSpec((H, H), const),
            pl.BlockSpec((1, H), const),
            pl.BlockSpec((1, H), const),
            pl.BlockSpec((1, H), const),
        ],
        out_specs=pl.BlockSpec((_RA, H), row),
        out_shape=jax.ShapeDtypeStruct((B * S, H), jnp.bfloat16),
    )(x2, pos_t, wp, bp, wc1t, bc1, wc2t, bc2, wq, bq, wk, bk, wv, bv,
      wo, bo, g, b)


_KC = 2048
_NK = (S * H) // _KC


def _pool_body(hs_ref, wa1_ref, ba1_ref, wa2_ref, ba2_ref, wr_ref, br_ref,
               out_ref, lg_ref, acc):
    k = pl.program_id(0)

    @pl.when(k == 0)
    def _():
        acc[...] = jnp.zeros_like(acc)

    acc[...] += _bdot(hs_ref[...], wa1_ref[...])

    @pl.when(k == _NK - 1)
    def _():
        u = jax.nn.gelu(acc[...] + ba1_ref[...])
        h0 = _bdot(u, wa2_ref[...]) + ba2_ref[...]
        out_ref[...] = h0
        lg_ref[...] = h0 @ wr_ref[...] + br_ref[...]


def _pool(hs_big, wa1, ba1, wa2, ba2, wr0, br0):
    c2 = lambda k: (0, 0)
    return pl.pallas_call(
        _pool_body,
        grid=(_NK,),
        in_specs=[
            pl.BlockSpec((B, _KC), lambda k: (0, k)),
            pl.BlockSpec((_KC, 2 * H), lambda k: (k, 0)),
            pl.BlockSpec((1, 2 * H), c2),
            pl.BlockSpec((2 * H, H), c2),
            pl.BlockSpec((1, H), c2),
            pl.BlockSpec((H, E), c2),
            pl.BlockSpec((1, E), c2),
        ],
        out_specs=[pl.BlockSpec((B, H), c2), pl.BlockSpec((B, E), c2)],
        out_shape=[jax.ShapeDtypeStruct((B, H), jnp.float32),
                   jax.ShapeDtypeStruct((B, E), jnp.float32)],
        scratch_shapes=[pltpu.VMEM((B, 2 * H), jnp.float32)],
    )(hs_big, wa1, ba1, wa2, ba2, wr0, br0)


_NW = 32
_TPW = B // _NW
_LN = 16


def _route_body(lt_hbm, out_hbm, lt_v, out_v):
    wid = lax.axis_index("s") * 2 + lax.axis_index("c")
    pltpu.sync_copy(lt_hbm.at[wid], lt_v)
    for ch in range(_TPW // _LN):
        sl = pl.ds(ch * _LN, _LN)
        v = [lt_v[e, sl] for e in range(E)]
        m = v[0]
        for e in range(1, E):
            m = jnp.maximum(m, v[e])
        ex = [jnp.exp(u - m) for u in v]
        tot = ex[0]
        for e in range(1, E):
            tot = tot + ex[e]
        p = [u / tot for u in ex]
        active = [jnp.full((_LN,), 1.0, jnp.float32) for _ in range(E)]
        acc = [jnp.zeros((_LN,), jnp.float32) for _ in range(E)]
        denom = jnp.zeros((_LN,), jnp.float32)
        for _ in range(K):
            cur = [p[e] * active[e] + (active[e] - 1.0) for e in range(E)]
            mv = cur[0]
            for e in range(1, E):
                mv = jnp.maximum(mv, cur[e])
            found = jnp.zeros((_LN,), jnp.float32)
            for e in range(E):
                eqf = jnp.where(cur[e] == mv, 1.0, 0.0)
                self_f = eqf * (1.0 - found)
                add = self_f * p[e]
                acc[e] = acc[e] + add
                denom = denom + add
                found = found + self_f
                active[e] = active[e] * (1.0 - self_f)
        for e in range(E):
            out_v[e, sl] = acc[e] / denom
    pltpu.sync_copy(out_v, out_hbm.at[wid])


def _route(ltw):
    mesh = plsc.VectorSubcoreMesh(core_axis_name="c", subcore_axis_name="s")
    return pl.kernel(
        _route_body,
        mesh=mesh,
        out_type=jax.ShapeDtypeStruct((_NW, E, _TPW), jnp.float32),
        scratch_types=[pltpu.VMEM((E, _TPW), jnp.float32),
                       pltpu.VMEM((E, _TPW), jnp.float32)],
    )(ltw)


def _moe_body(h_ref, comb_ref, we1_ref, be1_ref, we2_ref, be2_ref,
              g_ref, b_ref, wrn_ref, brn_ref, out_ref, lg_ref, moe):
    e = pl.program_id(0)
    lane = lax.broadcasted_iota(jnp.int32, (B, E), 1)

    @pl.when(e == 0)
    def _():
        moe[...] = jnp.zeros_like(moe)

    eh = jax.nn.gelu(_bdot(h_ref[...], we1_ref[0]) + be1_ref[0])
    eo = _bdot(eh, we2_ref[0]) + be2_ref[0]
    ce = jnp.sum(jnp.where(lane == e, comb_ref[...], 0.0), axis=-1,
                 keepdims=True)
    moe[...] += ce * eo

    @pl.when(e == E - 1)
    def _():
        z = h_ref[...] + moe[...]
        mu = jnp.mean(z, axis=-1, keepdims=True)
        var = jnp.mean((z - mu) ** 2, axis=-1, keepdims=True)
        hn = g_ref[...] * (z - mu) / jnp.sqrt(var + 1e-5) + b_ref[...]
        out_ref[...] = hn
        lg_ref[...] = hn @ wrn_ref[...] + brn_ref[...]


def _moe_layer(h, comb, we1, be1, we2, be2, g, b, wr_next, br_next):
    c2 = lambda e: (0, 0)
    return pl.pallas_call(
        _moe_body,
        grid=(E,),
        in_specs=[
            pl.BlockSpec((B, H), c2),
            pl.BlockSpec((B, E), c2),
            pl.BlockSpec((1, H, DFF), lambda e: (e, 0, 0)),
            pl.BlockSpec((1, 1, DFF), lambda e: (e, 0, 0)),
            pl.BlockSpec((1, DFF, H), lambda e: (e, 0, 0)),
            pl.BlockSpec((1, 1, H), lambda e: (e, 0, 0)),
            pl.BlockSpec((1, H), c2),
            pl.BlockSpec((1, H), c2),
            pl.BlockSpec((H, E), c2),
            pl.BlockSpec((1, E), c2),
        ],
        out_specs=[pl.BlockSpec((B, H), c2), pl.BlockSpec((B, E), c2)],
        out_shape=[jax.ShapeDtypeStruct((B, H), jnp.float32),
                   jax.ShapeDtypeStruct((B, E), jnp.float32)],
        scratch_shapes=[pltpu.VMEM((B, H), jnp.float32)],
    )(h, comb, we1, be1, we2, be2, g, b, wr_next, br_next)


_BTD = 256


def _head_body(h_ref, wexp_ref, bexp_ref, wh1_ref, bh1_ref, wh2_ref,
               bh2_ref, out_ref):
    d = jax.nn.gelu(_bdot(h_ref[...], wexp_ref[...]) + bexp_ref[...])
    d2 = d.reshape(_BTD * P, H)
    t = jax.nn.gelu(_bdot(d2, wh1_ref[...]) + bh1_ref[...])
    out_ref[...] = (jnp.sum(t * wh2_ref[...], axis=-1, keepdims=True)
                    + bh2_ref[...])


def _head(h, wexp, bexp, wh1, bh1, wh2t, bh2):
    n = B // _BTD
    c2 = lambda i: (0, 0)
    return pl.pallas_call(
        _head_body,
        grid=(n,),
        in_specs=[
            pl.BlockSpec((_BTD, H), lambda i: (i, 0)),
            pl.BlockSpec((H, P * H), c2),
            pl.BlockSpec((1, P * H), c2),
            pl.BlockSpec((H, H // 2), c2),
            pl.BlockSpec((1, H // 2), c2),
            pl.BlockSpec((1, H // 2), c2),
            pl.BlockSpec((1, 1), c2),
        ],
        out_specs=pl.BlockSpec((_BTD * P, 1), lambda i: (i, 0)),
        out_shape=jax.ShapeDtypeStruct((B * P, 1), jnp.float32),
    )(h, wexp, bexp, wh1, bh1, wh2t, bh2)


def kernel(x, pos_emb, Wp, bp, Wc1, bc1, Wc2, bc2, Wq, bq, Wk, bk, Wv, bv,
           Wo, bo, ln1g, ln1b, Wa1, ba1, Wa2, ba2, Wr, br, We1, be1, We2,
           be2, lng, lnb, Wexp, bexp, Wh1, bh1, Wh2, bh2):
    x2 = x.reshape(B * S, DIN)
    pos_t = jnp.tile(pos_emb[0], (_BTA, 1))
    wc1t = jnp.transpose(Wc1, (2, 1, 0))
    wc2t = jnp.transpose(Wc2, (2, 1, 0))
    r1 = lambda a: a.reshape(1, -1)

    hs = _encoder(x2, pos_t, Wp, r1(bp), wc1t, r1(bc1), wc2t, r1(bc2),
                  Wq, r1(bq), Wk, r1(bk), Wv, r1(bv), Wo, r1(bo),
                  r1(ln1g), r1(ln1b))
    h, lg = _pool(hs.reshape(B, S * H), Wa1, r1(ba1), Wa2, r1(ba2),
                  Wr[0], r1(br[0]))
    for l in range(L):
        ltw = lg.reshape(_NW, _TPW, E).transpose(0, 2, 1)
        cw = _route(ltw)
        comb = cw.transpose(0, 2, 1).reshape(B, E)
        ln = (l + 1) % L
        h, lg = _moe_layer(h, comb, We1[l], be1[l].reshape(E, 1, DFF),
                           We2[l], be2[l].reshape(E, 1, H), r1(lng[l]),
                           r1(lnb[l]), Wr[ln], r1(br[ln]))
    out = _head(h, Wexp, r1(bexp), Wh1, r1(bh1),
                Wh2.reshape(1, H // 2), bh2.reshape(1, 1))
    return out.reshape(B, P)


# speedup vs baseline: 1.0871x; 1.0484x over previous
import functools

import jax
import jax.numpy as jnp
import numpy as np
from jax import lax
from jax.experimental import pallas as pl
from jax.experimental.pallas import tpu as pltpu
from jax.experimental.pallas import tpu_sc as plsc

B = 1024
S = 96
P = 24
DIN = 64
H = 256
E = 8
K = 4
L = 4
NH = 4
DH = H // NH
DFF = 4 * H


_BTA = 32
_RA = _BTA * S


def _bdot(a, b):
    return jnp.dot(a.astype(jnp.bfloat16), b.astype(jnp.bfloat16),
                   preferred_element_type=jnp.float32)


def _encoder_body(x_ref, pos_ref, wp_ref, bp_ref, wc1_ref, bc1_ref,
                  wc2_ref, bc2_ref, wq_ref, bq_ref, wk_ref, bk_ref,
                  wv_ref, bv_ref, wo_ref, bo_ref, g_ref, b_ref, out_ref):
    f32 = jnp.float32
    t_iota = lax.broadcasted_iota(jnp.int32, (_RA, 1), 0) % S
    h = x_ref[...] @ wp_ref[...] + bp_ref[...] + pos_ref[...]

    def conv(u, wr, br):
        z0 = u @ wr[0]
        z1 = u @ wr[1]
        z2 = u @ wr[2]
        c = z1 + br[...]
        down = jnp.concatenate([jnp.zeros((1, H), f32), z0[:-1, :]], axis=0)
        c = c + jnp.where(t_iota > 0, down, 0.0)
        up = jnp.concatenate([z2[1:, :], jnp.zeros((1, H), f32)], axis=0)
        c = c + jnp.where(t_iota < S - 1, up, 0.0)
        return c

    c = jax.nn.gelu(conv(h, wc1_ref, bc1_ref))
    c = conv(c, wc2_ref, bc2_ref)
    h = h + c

    q = h @ wq_ref[...] + bq_ref[...]
    k = h @ wk_ref[...] + bk_ref[...]
    v = h @ wv_ref[...] + bv_ref[...]
    att = bo_ref[...]
    for n in range(NH):
        sl = slice(n * DH, (n + 1) * DH)
        qn = q[:, sl].reshape(_BTA, S, DH)
        kn = k[:, sl].reshape(_BTA, S, DH)
        vn = v[:, sl].reshape(_BTA, S, DH)
        s = lax.dot_general(qn, kn, (((2,), (2,)), ((0,), (0,))))
        a = jax.nn.softmax(s * (1.0 / 8.0), axis=-1)
        on = lax.dot_general(a, vn, (((2,), (1,)), ((0,), (0,))))
        att = att + on.reshape(_RA, DH) @ wo_ref[sl, :]
    z = h + att
    mu = jnp.mean(z, axis=-1, keepdims=True)
    var = jnp.mean((z - mu) ** 2, axis=-1, keepdims=True)
    hs = g_ref[...] * (z - mu) / jnp.sqrt(var + 1e-5) + b_ref[...]
    out_ref[...] = hs.astype(jnp.bfloat16)


def _encoder(x2, pos_t, wp, bp, wc1t, bc1, wc2t, bc2, wq, bq, wk, bk,
             wv, bv, wo, bo, g, b):
    n = B // _BTA
    row = lambda i: (i, 0)
    const = lambda i: (0, 0)
    const3 = lambda i: (0, 0, 0)
    return pl.pallas_call(
        _encoder_body,
        grid=(n,),
        in_specs=[
            pl.BlockSpec((_RA, DIN), row),
            pl.BlockSpec((_RA, H), const),
            pl.BlockSpec((DIN, H), const),
            pl.BlockSpec((1, H), const),
            pl.BlockSpec((3, H, H), const3),
            pl.BlockSpec((1, H), const),
            pl.BlockSpec((3, H, H), const3),
            pl.BlockSpec((1, H), const),
            pl.BlockSpec((H, H), const),
            pl.BlockSpec((1, H), const),
            pl.BlockSpec((H, H), const),
            pl.BlockSpec((1, H), const),
            pl.BlockSpec((H, H), const),
            pl.BlockSpec((1, H), const),
            pl.BlockSpec((H, H), const),
            pl.BlockSpec((1, H), const),
            pl.BlockSpec((1, H), const),
            pl.BlockSpec((1, H), const),
        ],
        out_specs=pl.BlockSpec((_RA, H), row),
        out_shape=jax.ShapeDtypeStruct((B * S, H), jnp.bfloat16),
    )(x2, pos_t, wp, bp, wc1t, bc1, wc2t, bc2, wq, bq, wk, bk, wv, bv,
      wo, bo, g, b)


_KC = 2048
_NK = (S * H) // _KC


def _pool_body(hs_ref, wa1_ref, ba1_ref, wa2_ref, ba2_ref, wr_ref, br_ref,
               out_ref, lg_ref, acc):
    k = pl.program_id(0)

    @pl.when(k == 0)
    def _():
        acc[...] = jnp.zeros_like(acc)

    acc[...] += _bdot(hs_ref[...], wa1_ref[...])

    @pl.when(k == _NK - 1)
    def _():
        u = jax.nn.gelu(acc[...] + ba1_ref[...])
        h0 = _bdot(u, wa2_ref[...]) + ba2_ref[...]
        out_ref[...] = h0
        lg_ref[...] = h0 @ wr_ref[...] + br_ref[...]


def _pool(hs_big, wa1, ba1, wa2, ba2, wr0, br0):
    c2 = lambda k: (0, 0)
    return pl.pallas_call(
        _pool_body,
        grid=(_NK,),
        in_specs=[
            pl.BlockSpec((B, _KC), lambda k: (0, k)),
            pl.BlockSpec((_KC, 2 * H), lambda k: (k, 0)),
            pl.BlockSpec((1, 2 * H), c2),
            pl.BlockSpec((2 * H, H), c2),
            pl.BlockSpec((1, H), c2),
            pl.BlockSpec((H, E), c2),
            pl.BlockSpec((1, E), c2),
        ],
        out_specs=[pl.BlockSpec((B, H), c2), pl.BlockSpec((B, E), c2)],
        out_shape=[jax.ShapeDtypeStruct((B, H), jnp.float32),
                   jax.ShapeDtypeStruct((B, E), jnp.float32)],
        scratch_shapes=[pltpu.VMEM((B, 2 * H), jnp.float32)],
    )(hs_big, wa1, ba1, wa2, ba2, wr0, br0)


_NW = 32
_TPW = B // _NW
_LN = 16


def _route_body(lt_hbm, out_hbm, lt_v, out_v):
    wid = lax.axis_index("s") * 2 + lax.axis_index("c")
    pltpu.sync_copy(lt_hbm.at[wid], lt_v)
    for ch in range(_TPW // _LN):
        sl = pl.ds(ch * _LN, _LN)
        v = [lt_v[e, sl] for e in range(E)]
        m = v[0]
        for e in range(1, E):
            m = jnp.maximum(m, v[e])
        ex = [jnp.exp(u - m) for u in v]
        tot = ex[0]
        for e in range(1, E):
            tot = tot + ex[e]
        p = [u / tot for u in ex]
        active = [jnp.full((_LN,), 1.0, jnp.float32) for _ in range(E)]
        acc = [jnp.zeros((_LN,), jnp.float32) for _ in range(E)]
        denom = jnp.zeros((_LN,), jnp.float32)
        for _ in range(K):
            cur = [p[e] * active[e] + (active[e] - 1.0) for e in range(E)]
            mv = cur[0]
            for e in range(1, E):
                mv = jnp.maximum(mv, cur[e])
            found = jnp.zeros((_LN,), jnp.float32)
            for e in range(E):
                eqf = jnp.where(cur[e] == mv, 1.0, 0.0)
                self_f = eqf * (1.0 - found)
                add = self_f * p[e]
                acc[e] = acc[e] + add
                denom = denom + add
                found = found + self_f
                active[e] = active[e] * (1.0 - self_f)
        for e in range(E):
            out_v[e, sl] = acc[e] / denom
    pltpu.sync_copy(out_v, out_hbm.at[wid])


def _route(ltw):
    mesh = plsc.VectorSubcoreMesh(core_axis_name="c", subcore_axis_name="s")
    return pl.kernel(
        _route_body,
        mesh=mesh,
        out_type=jax.ShapeDtypeStruct((_NW, E, _TPW), jnp.float32),
        scratch_types=[pltpu.VMEM((E, _TPW), jnp.float32),
                       pltpu.VMEM((E, _TPW), jnp.float32)],
    )(ltw)


def _moe_body(h_ref, comb_ref, we1_ref, be1_ref, we2_ref, be2_ref,
              g_ref, b_ref, wrn_ref, brn_ref, out_ref, lg_ref, moe):
    e = pl.program_id(0)
    lane = lax.broadcasted_iota(jnp.int32, (B, E), 1)

    @pl.when(e == 0)
    def _():
        moe[...] = jnp.zeros_like(moe)

    eh = jax.nn.gelu(_bdot(h_ref[...], we1_ref[0]) + be1_ref[0])
    eo = _bdot(eh, we2_ref[0]) + be2_ref[0]
    ce = jnp.sum(jnp.where(lane == e, comb_ref[...], 0.0), axis=-1,
                 keepdims=True)
    moe[...] += ce * eo

    @pl.when(e == E - 1)
    def _():
        z = h_ref[...] + moe[...]
        mu = jnp.mean(z, axis=-1, keepdims=True)
        var = jnp.mean((z - mu) ** 2, axis=-1, keepdims=True)
        hn = g_ref[...] * (z - mu) / jnp.sqrt(var + 1e-5) + b_ref[...]
        out_ref[...] = hn
        lg_ref[...] = hn @ wrn_ref[...] + brn_ref[...]


def _moe_layer(h, comb, we1, be1, we2, be2, g, b, wr_next, br_next):
    c2 = lambda e: (0, 0)
    return pl.pallas_call(
        _moe_body,
        grid=(E,),
        in_specs=[
            pl.BlockSpec((B, H), c2),
            pl.BlockSpec((B, E), c2),
            pl.BlockSpec((1, H, DFF), lambda e: (e, 0, 0)),
            pl.BlockSpec((1, 1, DFF), lambda e: (e, 0, 0)),
            pl.BlockSpec((1, DFF, H), lambda e: (e, 0, 0)),
            pl.BlockSpec((1, 1, H), lambda e: (e, 0, 0)),
            pl.BlockSpec((1, H), c2),
            pl.BlockSpec((1, H), c2),
            pl.BlockSpec((H, E), c2),
            pl.BlockSpec((1, E), c2),
        ],
        out_specs=[pl.BlockSpec((B, H), c2), pl.BlockSpec((B, E), c2)],
        out_shape=[jax.ShapeDtypeStruct((B, H), jnp.float32),
                   jax.ShapeDtypeStruct((B, E), jnp.float32)],
        scratch_shapes=[pltpu.VMEM((B, H), jnp.float32)],
    )(h, comb, we1, be1, we2, be2, g, b, wr_next, br_next)


_BTD = 256


def _head_body(h_ref, wexp_ref, bexp_ref, wh1_ref, bh1_ref, wh2_ref,
               bh2_ref, out_ref):
    d = jax.nn.gelu(_bdot(h_ref[...], wexp_ref[...]) + bexp_ref[...])
    d2 = d.reshape(_BTD * P, H)
    t = jax.nn.gelu(_bdot(d2, wh1_ref[...]) + bh1_ref[...])
    out_ref[...] = (jnp.sum(t * wh2_ref[...], axis=-1, keepdims=True)
                    + bh2_ref[...])


def _head(h, wexp, bexp, wh1, bh1, wh2t, bh2):
    n = B // _BTD
    c2 = lambda i: (0, 0)
    return pl.pallas_call(
        _head_body,
        grid=(n,),
        in_specs=[
            pl.BlockSpec((_BTD, H), lambda i: (i, 0)),
            pl.BlockSpec((H, P * H), c2),
            pl.BlockSpec((1, P * H), c2),
            pl.BlockSpec((H, H // 2), c2),
            pl.BlockSpec((1, H // 2), c2),
            pl.BlockSpec((1, H // 2), c2),
            pl.BlockSpec((1, 1), c2),
        ],
        out_specs=pl.BlockSpec((_BTD * P, 1), lambda i: (i, 0)),
        out_shape=jax.ShapeDtypeStruct((B * P, 1), jnp.float32),
    )(h, wexp, bexp, wh1, bh1, wh2t, bh2)


def kernel(x, pos_emb, Wp, bp, Wc1, bc1, Wc2, bc2, Wq, bq, Wk, bk, Wv, bv,
           Wo, bo, ln1g, ln1b, Wa1, ba1, Wa2, ba2, Wr, br, We1, be1, We2,
           be2, lng, lnb, Wexp, bexp, Wh1, bh1, Wh2, bh2):
    x2 = x.reshape(B * S, DIN)
    pos_t = jnp.tile(pos_emb[0], (_BTA, 1))
    wc1t = jnp.transpose(Wc1, (2, 1, 0))
    wc2t = jnp.transpose(Wc2, (2, 1, 0))
    r1 = lambda a: a.reshape(1, -1)

    hs = _encoder(x2, pos_t, Wp, r1(bp), wc1t, r1(bc1), wc2t, r1(bc2),
                  Wq, r1(bq), Wk, r1(bk), Wv, r1(bv), Wo, r1(bo),
                  r1(ln1g), r1(ln1b))
    h, lg = _pool(hs.reshape(B, S * H), Wa1, r1(ba1), Wa2, r1(ba2),
                  Wr[0], r1(br[0]))
    for l in range(L):
        ltw = lg.reshape(_NW, _TPW, E).transpose(0, 2, 1)
        cw = _route(ltw)
        comb = cw.transpose(0, 2, 1).reshape(B, E)
        ln = (l + 1) % L
        h, lg = _moe_layer(h, comb, We1[l], be1[l].reshape(E, 1, DFF),
                           We2[l], be2[l].reshape(E, 1, H), r1(lng[l]),
                           r1(lnb[l]), Wr[ln], r1(br[ln]))
    out = _head(h, Wexp, r1(bexp), Wh1, r1(bh1),
                Wh2.reshape(1, H // 2), bh2.reshape(1, 1))
    return out.reshape(B, P)


# speedup vs baseline: 1.0905x; 1.0032x over previous
import jax
import jax.numpy as jnp
from jax import lax
from jax.experimental import pallas as pl
from jax.experimental.pallas import tpu as pltpu
from jax.experimental.pallas import tpu_sc as plsc

B = 1024
S = 96
P = 24
DIN = 64
H = 256
E = 8
K = 4
L = 4
NH = 4
DH = H // NH
DFF = 4 * H


_BTA = 32
_RA = _BTA * S


def _bdot(a, b):
    return jnp.dot(a.astype(jnp.bfloat16), b.astype(jnp.bfloat16),
                   preferred_element_type=jnp.float32)


def _encoder_body(x_ref, pos_ref, wp_ref, bp_ref, wc1_ref, bc1_ref,
                  wc2_ref, bc2_ref, wq_ref, bq_ref, wk_ref, bk_ref,
                  wv_ref, bv_ref, wo_ref, bo_ref, g_ref, b_ref, out_ref):
    f32 = jnp.float32
    t_iota = lax.broadcasted_iota(jnp.int32, (_RA, 1), 0) % S
    h = x_ref[...] @ wp_ref[...] + bp_ref[...] + pos_ref[...]

    def conv(u, wr, br):
        z0 = u @ wr[0]
        z1 = u @ wr[1]
        z2 = u @ wr[2]
        c = z1 + br[...]
        down = jnp.concatenate([jnp.zeros((1, H), f32), z0[:-1, :]], axis=0)
        c = c + jnp.where(t_iota > 0, down, 0.0)
        up = jnp.concatenate([z2[1:, :], jnp.zeros((1, H), f32)], axis=0)
        c = c + jnp.where(t_iota < S - 1, up, 0.0)
        return c

    c = jax.nn.gelu(conv(h, wc1_ref, bc1_ref))
    c = conv(c, wc2_ref, bc2_ref)
    h = h + c

    q = h @ wq_ref[...] + bq_ref[...]
    k = h @ wk_ref[...] + bk_ref[...]
    v = h @ wv_ref[...] + bv_ref[...]
    att = bo_ref[...]
    for n in range(NH):
        sl = slice(n * DH, (n + 1) * DH)
        qn = q[:, sl].reshape(_BTA, S, DH)
        kn = k[:, sl].reshape(_BTA, S, DH)
        vn = v[:, sl].reshape(_BTA, S, DH)
        s = lax.dot_general(qn, kn, (((2,), (2,)), ((0,), (0,))))
        a = jax.nn.softmax(s * (1.0 / 8.0), axis=-1)
        on = lax.dot_general(a, vn, (((2,), (1,)), ((0,), (0,))))
        att = att + on.reshape(_RA, DH) @ wo_ref[sl, :]
    z = h + att
    mu = jnp.mean(z, axis=-1, keepdims=True)
    var = jnp.mean((z - mu) ** 2, axis=-1, keepdims=True)
    hs = g_ref[...] * (z - mu) / jnp.sqrt(var + 1e-5) + b_ref[...]
    out_ref[...] = hs.astype(jnp.bfloat16)


def _encoder(x2, pos_t, wp, bp, wc1t, bc1, wc2t, bc2, wq, bq, wk, bk,
             wv, bv, wo, bo, g, b):
    n = B // _BTA
    row = lambda i: (i, 0)
    const = lambda i: (0, 0)
    const3 = lambda i: (0, 0, 0)
    return pl.pallas_call(
        _encoder_body,
        grid=(n,),
        in_specs=[
            pl.BlockSpec((_RA, DIN), row),
            pl.BlockSpec((_RA, H), const),
            pl.BlockSpec((DIN, H), const),
            pl.BlockSpec((1, H), const),
            pl.BlockSpec((3, H, H), const3),
            pl.BlockSpec((1, H), const),
            pl.BlockSpec((3, H, H), const3),
            pl.BlockSpec((1, H), const),
            pl.BlockSpec((H, H), const),
            pl.BlockSpec((1, H), const),
            pl.BlockSpec((H, H), const),
            pl.BlockSpec((1, H), const),
            pl.BlockSpec((H, H), const),
            pl.BlockSpec((1, H), const),
            pl.BlockSpec((H, H), const),
            pl.BlockSpec((1, H), const),
            pl.BlockSpec((1, H), const),
            pl.BlockSpec((1, H), const),
        ],
        out_specs=pl.BlockSpec((_RA, H), row),
        out_shape=jax.ShapeDtypeStruct((B * S, H), jnp.bfloat16),
    )(x2, pos_t, wp, bp, wc1t, bc1, wc2t, bc2, wq, bq, wk, bk, wv, bv,
      wo, bo, g, b)


_KC = 2048
_NK = (S * H) // _KC


def _pool_body(hs_ref, wa1_ref, ba1_ref, wa2_ref, ba2_ref, wr_ref, br_ref,
               out_ref, lg_ref, acc):
    k = pl.program_id(0)

    @pl.when(k == 0)
    def _():
        acc[...] = jnp.zeros_like(acc)

    acc[...] += _bdot(hs_ref[...], wa1_ref[...])

    @pl.when(k == _NK - 1)
    def _():
        u = jax.nn.gelu(acc[...] + ba1_ref[...])
        h0 = _bdot(u, wa2_ref[...]) + ba2_ref[...]
        out_ref[...] = h0
        lg_ref[...] = h0 @ wr_ref[...] + br_ref[...]


def _pool(hs_big, wa1, ba1, wa2, ba2, wr0, br0):
    c2 = lambda k: (0, 0)
    return pl.pallas_call(
        _pool_body,
        grid=(_NK,),
        in_specs=[
            pl.BlockSpec((B, _KC), lambda k: (0, k)),
            pl.BlockSpec((_KC, 2 * H), lambda k: (k, 0)),
            pl.BlockSpec((1, 2 * H), c2),
            pl.BlockSpec((2 * H, H), c2),
            pl.BlockSpec((1, H), c2),
            pl.BlockSpec((H, E), c2),
            pl.BlockSpec((1, E), c2),
        ],
        out_specs=[pl.BlockSpec((B, H), c2), pl.BlockSpec((B, E), c2)],
        out_shape=[jax.ShapeDtypeStruct((B, H), jnp.float32),
                   jax.ShapeDtypeStruct((B, E), jnp.float32)],
        scratch_shapes=[pltpu.VMEM((B, 2 * H), jnp.float32)],
    )(hs_big, wa1, ba1, wa2, ba2, wr0, br0)


_NW = 32
_TPW = B // _NW
_LN = 16


def _route_body(lt_hbm, out_hbm, lt_v, out_v):
    wid = lax.axis_index("s") * 2 + lax.axis_index("c")
    pltpu.sync_copy(lt_hbm.at[wid], lt_v)
    for ch in range(_TPW // _LN):
        sl = pl.ds(ch * _LN, _LN)
        v = [lt_v[e, sl] for e in range(E)]
        m = v[0]
        for e in range(1, E):
            m = jnp.maximum(m, v[e])
        ex = [jnp.exp(u - m) for u in v]
        tot = ex[0]
        for e in range(1, E):
            tot = tot + ex[e]
        p = [u / tot for u in ex]
        active = [jnp.full((_LN,), 1.0, jnp.float32) for _ in range(E)]
        acc = [jnp.zeros((_LN,), jnp.float32) for _ in range(E)]
        denom = jnp.zeros((_LN,), jnp.float32)
        for _ in range(K):
            cur = [p[e] * active[e] + (active[e] - 1.0) for e in range(E)]
            mv = cur[0]
            for e in range(1, E):
                mv = jnp.maximum(mv, cur[e])
            found = jnp.zeros((_LN,), jnp.float32)
            for e in range(E):
                eqf = jnp.where(cur[e] == mv, 1.0, 0.0)
                self_f = eqf * (1.0 - found)
                add = self_f * p[e]
                acc[e] = acc[e] + add
                denom = denom + add
                found = found + self_f
                active[e] = active[e] * (1.0 - self_f)
        for e in range(E):
            out_v[e, sl] = acc[e] / denom
    pltpu.sync_copy(out_v, out_hbm.at[wid])


def _route(ltw):
    mesh = plsc.VectorSubcoreMesh(core_axis_name="c", subcore_axis_name="s")
    return pl.kernel(
        _route_body,
        mesh=mesh,
        out_type=jax.ShapeDtypeStruct((_NW, E, _TPW), jnp.float32),
        scratch_types=[pltpu.VMEM((E, _TPW), jnp.float32),
                       pltpu.VMEM((E, _TPW), jnp.float32)],
    )(ltw)


def _moe_body(h_ref, comb_ref, we1_ref, be1_ref, we2_ref, be2_ref,
              g_ref, b_ref, wrn_ref, brn_ref, out_ref, lg_ref, moe):
    e = pl.program_id(0)
    lane = lax.broadcasted_iota(jnp.int32, (B, E), 1)

    @pl.when(e == 0)
    def _():
        moe[...] = jnp.zeros_like(moe)

    eh = jax.nn.gelu(_bdot(h_ref[...], we1_ref[0]) + be1_ref[0])
    eo = _bdot(eh, we2_ref[0]) + be2_ref[0]
    ce = jnp.sum(jnp.where(lane == e, comb_ref[...], 0.0), axis=-1,
                 keepdims=True)
    moe[...] += ce * eo

    @pl.when(e == E - 1)
    def _():
        z = h_ref[...] + moe[...]
        mu = jnp.mean(z, axis=-1, keepdims=True)
        var = jnp.mean((z - mu) ** 2, axis=-1, keepdims=True)
        hn = g_ref[...] * (z - mu) / jnp.sqrt(var + 1e-5) + b_ref[...]
        out_ref[...] = hn
        lg_ref[...] = hn @ wrn_ref[...] + brn_ref[...]


def _moe_layer(h, comb, we1, be1, we2, be2, g, b, wr_next, br_next):
    c2 = lambda e: (0, 0)
    return pl.pallas_call(
        _moe_body,
        grid=(E,),
        in_specs=[
            pl.BlockSpec((B, H), c2),
            pl.BlockSpec((B, E), c2),
            pl.BlockSpec((1, H, DFF), lambda e: (e, 0, 0)),
            pl.BlockSpec((1, 1, DFF), lambda e: (e, 0, 0)),
            pl.BlockSpec((1, DFF, H), lambda e: (e, 0, 0)),
            pl.BlockSpec((1, 1, H), lambda e: (e, 0, 0)),
            pl.BlockSpec((1, H), c2),
            pl.BlockSpec((1, H), c2),
            pl.BlockSpec((H, E), c2),
            pl.BlockSpec((1, E), c2),
        ],
        out_specs=[pl.BlockSpec((B, H), c2), pl.BlockSpec((B, E), c2)],
        out_shape=[jax.ShapeDtypeStruct((B, H), jnp.float32),
                   jax.ShapeDtypeStruct((B, E), jnp.float32)],
        scratch_shapes=[pltpu.VMEM((B, H), jnp.float32)],
    )(h, comb, we1, be1, we2, be2, g, b, wr_next, br_next)


_BTD = 256


def _head_body(h_ref, wexp_ref, bexp_ref, wh1_ref, bh1_ref, wh2_ref,
               bh2_ref, out_ref):
    d = jax.nn.gelu(_bdot(h_ref[...], wexp_ref[...]) + bexp_ref[...])
    d2 = d.reshape(_BTD * P, H)
    t = jax.nn.gelu(_bdot(d2, wh1_ref[...]) + bh1_ref[...])
    out_ref[...] = (jnp.sum(t * wh2_ref[...], axis=-1, keepdims=True)
                    + bh2_ref[...])


def _head(h, wexp, bexp, wh1, bh1, wh2t, bh2):
    n = B // _BTD
    c2 = lambda i: (0, 0)
    return pl.pallas_call(
        _head_body,
        grid=(n,),
        in_specs=[
            pl.BlockSpec((_BTD, H), lambda i: (i, 0)),
            pl.BlockSpec((H, P * H), c2),
            pl.BlockSpec((1, P * H), c2),
            pl.BlockSpec((H, H // 2), c2),
            pl.BlockSpec((1, H // 2), c2),
            pl.BlockSpec((1, H // 2), c2),
            pl.BlockSpec((1, 1), c2),
        ],
        out_specs=pl.BlockSpec((_BTD * P, 1), lambda i: (i, 0)),
        out_shape=jax.ShapeDtypeStruct((B * P, 1), jnp.float32),
    )(h, wexp, bexp, wh1, bh1, wh2t, bh2)


def kernel(x, pos_emb, Wp, bp, Wc1, bc1, Wc2, bc2, Wq, bq, Wk, bk, Wv, bv,
           Wo, bo, ln1g, ln1b, Wa1, ba1, Wa2, ba2, Wr, br, We1, be1, We2,
           be2, lng, lnb, Wexp, bexp, Wh1, bh1, Wh2, bh2):
    x2 = x.reshape(B * S, DIN)
    pos_t = jnp.tile(pos_emb[0], (_BTA, 1))
    wc1t = jnp.transpose(Wc1, (2, 1, 0))
    wc2t = jnp.transpose(Wc2, (2, 1, 0))
    r1 = lambda a: a.reshape(1, -1)

    hs = _encoder(x2, pos_t, Wp, r1(bp), wc1t, r1(bc1), wc2t, r1(bc2),
                  Wq, r1(bq), Wk, r1(bk), Wv, r1(bv), Wo, r1(bo),
                  r1(ln1g), r1(ln1b))
    h, lg = _pool(hs.reshape(B, S * H), Wa1, r1(ba1), Wa2, r1(ba2),
                  Wr[0], r1(br[0]))
    for l in range(L):
        ltw = lg.reshape(_NW, _TPW, E).transpose(0, 2, 1)
        cw = _route(ltw)
        comb = cw.transpose(0, 2, 1).reshape(B, E)
        ln = (l + 1) % L
        h, lg = _moe_layer(h, comb, We1[l], be1[l].reshape(E, 1, DFF),
                           We2[l], be2[l].reshape(E, 1, H), r1(lng[l]),
                           r1(lnb[l]), Wr[ln], r1(br[ln]))
    out = _head(h, Wexp, r1(bexp), Wh1, r1(bh1),
                Wh2.reshape(1, H // 2), bh2.reshape(1, 1))
    return out.reshape(B, P)
